# baseline (speedup 1.0000x reference)
import functools

import jax
import jax.numpy as jnp
from jax import lax
from jax.experimental import pallas as pl
from jax.experimental.pallas import tpu as pltpu
from jax.experimental.pallas import tpu_sc as plsc

BATCH = 16384
DIM = 32
ROWS = 1000000
GROW = 128
GROWS = ROWS // 4
NC = 2
NS = 16
NW = NC * NS
BPW = BATCH // NW
BLK = 256
NBLK = BPW // BLK
CHUNK = 128
NTC = ROWS // GROW
TPW = 245


def _detile_body(uwt_hbm, iwt_hbm, uwd_hbm, iwd_hbm,
                 cin_v, stg_v, sem_in, sem_out):
    wid = lax.axis_index("s") * NC + lax.axis_index("c")
    start = wid * TPW
    count = jnp.minimum(TPW, jnp.maximum(NTC - start, 0))
    lanes = lax.iota(jnp.int32, 16)

    def one_table(src_hbm, dst_hbm):
        def chunk(k, _):
            tc = start + k
            c0 = pl.multiple_of(tc * GROW, GROW)
            pltpu.async_copy(src_hbm.at[:, pl.ds(c0, GROW)], cin_v,
                             sem_in).wait()
            for g in range(32):
                for m in range(4):
                    col = jnp.full((16,), g * 4 + m, jnp.int32)
                    lo = plsc.load_gather(cin_v, [lanes, col])
                    hi = plsc.load_gather(cin_v, [lanes + 16, col])
                    o = m * DIM
                    stg_v[g, pl.ds(o, 16)] = lo
                    stg_v[g, pl.ds(o + 16, 16)] = hi
            r0 = pl.multiple_of(tc * 32, 8)
            pltpu.async_copy(stg_v, dst_hbm.at[pl.ds(r0, 32), :],
                             sem_out).wait()
            return _

        lax.fori_loop(0, count, chunk, 0)

    one_table(uwt_hbm, uwd_hbm)
    one_table(iwt_hbm, iwd_hbm)

    @pl.when(wid == 0)
    def _():
        c0 = pl.multiple_of(NTC * GROW, GROW)

        def tail(src_hbm, dst_hbm):
            pltpu.async_copy(src_hbm.at[:, pl.ds(c0, GROW)], cin_v,
                             sem_in).wait()
            for g in range(16):
                for m in range(4):
                    col = jnp.full((16,), g * 4 + m, jnp.int32)
                    lo = plsc.load_gather(cin_v, [lanes, col])
                    hi = plsc.load_gather(cin_v, [lanes + 16, col])
                    o = m * DIM
                    stg_v[g, pl.ds(o, 16)] = lo
                    stg_v[g, pl.ds(o + 16, 16)] = hi
            r0 = pl.multiple_of(NTC * 32, 8)
            pltpu.async_copy(stg_v.at[pl.ds(0, 16), :],
                             dst_hbm.at[pl.ds(r0, 16), :], sem_out).wait()

        tail(uwt_hbm, uwd_hbm)
        tail(iwt_hbm, iwd_hbm)


@jax.jit
def _mf(u, i, uwt, iwt):
    mesh = plsc.VectorSubcoreMesh(core_axis_name="c", subcore_axis_name="s")
    uwd, iwd = pl.kernel(
        _detile_body,
        out_type=(jax.ShapeDtypeStruct((GROWS, GROW), jnp.float32),
                  jax.ShapeDtypeStruct((GROWS, GROW), jnp.float32)),
        mesh=mesh,
        compiler_params=pltpu.CompilerParams(
            needs_layout_passes=False, use_tc_tiling_on_sc=True,
            disable_bounds_checks=True),
        scratch_types=[
            pltpu.VMEM((DIM, GROW), jnp.float32),
            pltpu.VMEM((32, GROW), jnp.float32),
            pltpu.SemaphoreType.DMA,
            pltpu.SemaphoreType.DMA,
        ],
    )(uwt, iwt)

    return pl.kernel(
        _dot_body,
        out_type=jax.ShapeDtypeStruct((BATCH,), jnp.float32),
        mesh=mesh,
        compiler_params=pltpu.CompilerParams(
            needs_layout_passes=False, use_tc_tiling_on_sc=True),
        scratch_types=[
            pltpu.VMEM((BLK,), jnp.int32),
            pltpu.VMEM((BLK,), jnp.int32),
            pltpu.VMEM((BLK,), jnp.int32),
            pltpu.VMEM((BLK,), jnp.int32),
            pltpu.VMEM((BLK, GROW), jnp.float32),
            pltpu.VMEM((BLK, GROW), jnp.float32),
            pltpu.VMEM((16 * 17,), jnp.float32),
            pltpu.VMEM((BLK,), jnp.float32),
            pltpu.SemaphoreType.DMA,
            pltpu.SemaphoreType.DMA,
        ],
    )(u >> 2, i >> 2, u & 3, i & 3, uwd, iwd)


def _dot_body(uq_hbm, iq_hbm, uo_hbm, io_hbm, uw_hbm, iw_hbm, out_hbm,
              uq_v, iq_v, uo_v, io_v, ue_v, ie_v, part_v, out_v,
              sem_u, sem_i):
    wid = lax.axis_index("s") * NC + lax.axis_index("c")
    base_b = wid * BPW
    lanes = lax.iota(jnp.int32, 16)

    for blk in range(NBLK):
        b0 = base_b + blk * BLK
        pltpu.sync_copy(uq_hbm.at[pl.ds(b0, BLK)], uq_v)
        pltpu.sync_copy(iq_hbm.at[pl.ds(b0, BLK)], iq_v)
        pltpu.sync_copy(uo_hbm.at[pl.ds(b0, BLK)], uo_v)
        pltpu.sync_copy(io_hbm.at[pl.ds(b0, BLK)], io_v)

        pending = []
        for c in range(BLK // CHUNK):
            o = c * CHUNK
            pending.append(pltpu.async_copy(
                uw_hbm.at[uq_v.at[pl.ds(o, CHUNK)]],
                ue_v.at[pl.ds(o, CHUNK)], sem_u))
            pending.append(pltpu.async_copy(
                iw_hbm.at[iq_v.at[pl.ds(o, CHUNK)]],
                ie_v.at[pl.ds(o, CHUNK)], sem_i))
        for p in pending:
            p.wait()

        def group(g, _):
            gb = g * 16
            um = uo_v[pl.ds(gb, 16)]
            im = io_v[pl.ds(gb, 16)]
            for r in range(16):
                uoff = um[r] * DIM
                ioff = im[r] * DIM
                p = (ue_v[gb + r, pl.ds(uoff, 16)]
                     * ie_v[gb + r, pl.ds(ioff, 16)]
                     + ue_v[gb + r, pl.ds(uoff + 16, 16)]
                     * ie_v[gb + r, pl.ds(ioff + 16, 16)])
                part_v[pl.ds(r * 17, 16)] = p
            acc = plsc.load_gather(part_v, [lanes * 17])
            for l in range(1, 16):
                acc = acc + plsc.load_gather(part_v, [lanes * 17 + l])
            out_v[pl.ds(gb, 16)] = acc
            return _

        lax.fori_loop(0, BLK // 16, group, 0)

        pltpu.sync_copy(out_v, out_hbm.at[pl.ds(b0, BLK)])


def kernel(u, i, user_weight, item_weight):
    return _mf(u, i, user_weight.T, item_weight.T)
